# baseline (speedup 1.0000x reference)
import functools

import jax
import jax.numpy as jnp
from jax import lax
from jax.experimental import pallas as pl
from jax.experimental.pallas import tpu as pltpu
from jax.experimental.pallas import tpu_sc as plsc

_NC, _NS = 2, 16
_NW = _NC * _NS
_NSPLIT = 4


def _make_sc_copy(max_pos, emb_dim, dtype):
    rows_per_w = max_pos // _NW
    chunk = rows_per_w // _NSPLIT
    mesh = plsc.VectorSubcoreMesh(core_axis_name="c", subcore_axis_name="s")

    @functools.partial(
        pl.kernel,
        mesh=mesh,
        out_type=jax.ShapeDtypeStruct((max_pos, emb_dim), dtype),
        scratch_types=[
            pltpu.SemaphoreType.DMA((_NSPLIT,)),
        ],
    )
    def sc_copy(pe_hbm, out_hbm, sems):
        wid = lax.axis_index("s") * _NC + lax.axis_index("c")
        base = wid * rows_per_w
        copies = [
            pltpu.async_copy(
                pe_hbm.at[pl.ds(base + g * chunk, chunk)],
                out_hbm.at[pl.ds(base + g * chunk, chunk)],
                sems.at[g],
            )
            for g in range(_NSPLIT)
        ]
        for c in copies:
            c.wait()

    return sc_copy


def kernel(x, pe_table):
    del x
    max_pos, emb_dim = pe_table.shape
    out = _make_sc_copy(max_pos, emb_dim, pe_table.dtype)(pe_table)
    return out[None]


# speedup vs baseline: 49.8053x; 49.8053x over previous
import jax
import jax.numpy as jnp
from jax.experimental import pallas as pl
from jax.experimental.pallas import tpu as pltpu

_CHUNK = 512
_NBUF = 8


def _make_body(n_chunks):
    def body(pe_hbm, o_hbm, buf, in_sems, out_sems):
        def src(g):
            return pe_hbm.at[pl.ds(g * _CHUNK, _CHUNK)]

        def dst(g):
            return o_hbm.at[pl.ds(g * _CHUNK, _CHUNK)]

        ins = {}
        outs = {}
        for b in range(min(_NBUF, n_chunks)):
            ins[b] = pltpu.make_async_copy(src(b), buf.at[b], in_sems.at[b])
            ins[b].start()
        for g in range(n_chunks):
            b = g % _NBUF
            ins[g].wait()
            outs[g] = pltpu.make_async_copy(buf.at[b], dst(g), out_sems.at[b])
            outs[g].start()
            ng = g + _NBUF
            if ng < n_chunks:
                outs[g].wait()
                ins[ng] = pltpu.make_async_copy(src(ng), buf.at[b], in_sems.at[b])
                ins[ng].start()
        for g in range(max(0, n_chunks - _NBUF), n_chunks):
            outs[g].wait()

    return body


def kernel(x, pe_table):
    del x
    max_pos, emb_dim = pe_table.shape
    n_chunks = max_pos // _CHUNK
    out = pl.pallas_call(
        _make_body(n_chunks),
        in_specs=[pl.BlockSpec(memory_space=pltpu.MemorySpace.HBM)],
        out_specs=pl.BlockSpec(memory_space=pltpu.MemorySpace.HBM),
        out_shape=jax.ShapeDtypeStruct((max_pos, emb_dim), pe_table.dtype),
        scratch_shapes=[
            pltpu.VMEM((_NBUF, _CHUNK, emb_dim), pe_table.dtype),
            pltpu.SemaphoreType.DMA((_NBUF,)),
            pltpu.SemaphoreType.DMA((_NBUF,)),
        ],
    )(pe_table)
    return out[None]
